# Initial kernel scaffold; baseline (speedup 1.0000x reference)
import jax
import jax.numpy as jnp
from jax.experimental import pallas as pl


def kernel(edge_index, vertex_features, batch, params):
    raise NotImplementedError("write your pallas kernel here")


# speedup vs baseline: 2.1671x; 2.1671x over previous
import functools

import jax
import jax.numpy as jnp
from jax import lax
from jax.experimental import pallas as pl
from jax.experimental.pallas import tpu as pltpu
from jax.experimental.pallas import tpu_sc as plsc

N = 10000
E = 320000
NUM_GRAPHS = 16

NC, NS, L = 2, 16, 16
NW = NC * NS
NRANGE = 313
NPAD = NW * NRANGE
SENT = NRANGE
CH = 8000
FBUF = CH + 128
RLEN = E + CH + 512

_MESH = plsc.VectorSubcoreMesh(
    core_axis_name="c", subcore_axis_name="s", num_cores=NC, num_subcores=NS
)
_SC_PARAMS = pltpu.CompilerParams(needs_layout_passes=False)

_BISECT = 1
_HIGH = jax.lax.Precision.HIGHEST


def _wid():
    return lax.axis_index("s") * NC + lax.axis_index("c")


def _bucket_body(src_hbm, dst_hbm, srcl, ldstl, cnts, s_st, d_st, fsrc, fldst,
                 cnt_v):
    wid = _wid()
    lo = wid * NRANGE
    hi = jnp.minimum(lo + NRANGE, N)
    base = wid * RLEN

    def fill_src(i, _):
        fsrc[pl.ds(i * L, L)] = jnp.zeros((L,), jnp.int32)
        return 0

    lax.fori_loop(0, FBUF // L, fill_src, 0)

    def chunk(ci, total):
        def fill_ld(i, _):
            fldst[pl.ds(i * L, L)] = jnp.full((L,), SENT, jnp.int32)
            return 0

        lax.fori_loop(0, FBUF // L, fill_ld, 0)
        pltpu.sync_copy(src_hbm.at[pl.ds(ci * CH, CH)], s_st)
        pltpu.sync_copy(dst_hbm.at[pl.ds(ci * CH, CH)], d_st)

        lanes = jnp.arange(L, dtype=jnp.int32)
        w1 = hi - lo - 1

        def scan16(j, off):
            d = d_st[pl.ds(j * L, L)]
            s = s_st[pl.ds(j * L, L)]
            u = d - lo
            g = jnp.minimum(u, w1 - u)
            inr = jnp.maximum(jnp.minimum(g, 0) + 1, 0)
            cs = plsc.cumsum(inr)
            pos = inr * (off + cs - 1) + (1 - inr) * (FBUF + lanes)
            plsc.store_scatter(fsrc, [pos], s)
            plsc.store_scatter(fldst, [pos], inr * u + (1 - inr) * SENT)
            return off + cs[L - 1]

        off = lax.fori_loop(0, CH // L, scan16, jnp.int32(0))
        fo = pl.multiple_of(base + total, 8)
        pltpu.sync_copy(fsrc.at[pl.ds(0, FBUF)], srcl.at[pl.ds(fo, FBUF)])
        pltpu.sync_copy(fldst.at[pl.ds(0, FBUF)], ldstl.at[pl.ds(fo, FBUF)])
        return total + ((off + 7) // 8) * 8

    total = lax.fori_loop(0, E // CH, chunk, jnp.int32(0))
    nb = (total + 127) // 128
    cnt_v[...] = jnp.full((L,), 0, jnp.int32) + nb
    pltpu.sync_copy(cnt_v, cnts.at[pl.ds(pl.multiple_of(wid * L, 8), L)])


_bucket = pl.kernel(
    _bucket_body,
    out_type=(
        jax.ShapeDtypeStruct((NW * RLEN,), jnp.int32),
        jax.ShapeDtypeStruct((NW * RLEN,), jnp.int32),
        jax.ShapeDtypeStruct((NW * L,), jnp.int32),
    ),
    mesh=_MESH,
    compiler_params=_SC_PARAMS,
    scratch_types=[
        pltpu.VMEM((CH,), jnp.int32),
        pltpu.VMEM((CH,), jnp.int32),
        pltpu.VMEM((FBUF + L,), jnp.int32),
        pltpu.VMEM((FBUF + L,), jnp.int32),
        pltpu.VMEM((L,), jnp.int32),
    ],
)


def _make_segmax(D):
    def body(y_hbm, srcl, ldstl, cnts, segout, idx_v, ldst_v, rows, seg, cnt_v,
             sem):
        wid = _wid()
        base = wid * RLEN

        def ini(t, _):
            seg[pl.ds(t * L, L)] = jnp.full((L,), -jnp.inf, jnp.float32)
            return 0

        lax.fori_loop(0, (NRANGE + 1) * D // L, ini, 0)
        pltpu.sync_copy(cnts.at[pl.ds(pl.multiple_of(wid * L, 8), L)], cnt_v)
        nb = jnp.max(cnt_v[...])

        def batch(b, _):
            bo = pl.multiple_of(base + b * 128, 8)
            pltpu.sync_copy(srcl.at[pl.ds(bo, 128)], idx_v)
            pltpu.sync_copy(ldstl.at[pl.ds(bo, 128)],
                            ldst_v.at[pl.ds(0, 128)])
            pltpu.async_copy(y_hbm.at[idx_v], rows, sem).wait()

            lanes = jnp.arange(L, dtype=jnp.int32)

            def edge(g, _):
                ld = ldst_v[pl.ds(g, L)][0]
                sb = ld * D
                gs = jnp.zeros((L,), jnp.int32) + g
                for c in range(D // L):
                    a = seg[pl.ds(sb + c * L, L)]
                    v = plsc.load_gather(rows, [gs, lanes + c * L])
                    seg[pl.ds(sb + c * L, L)] = jnp.maximum(a, v)
                return 0

            lax.fori_loop(0, 128, edge, 0)
            return 0

        lax.fori_loop(0, nb, batch, 0)
        pltpu.sync_copy(
            seg.at[pl.ds(0, NRANGE * D)],
            segout.at[pl.ds(pl.multiple_of(wid * (NRANGE * D), 8), NRANGE * D)])

    return pl.kernel(
        body,
        out_type=jax.ShapeDtypeStruct((NPAD * D,), jnp.float32),
        mesh=_MESH,
        compiler_params=_SC_PARAMS,
        scratch_types=[
            pltpu.VMEM((128,), jnp.int32),
            pltpu.VMEM((128 + L,), jnp.int32),
            pltpu.VMEM((128, D), jnp.float32),
            pltpu.VMEM(((NRANGE + 1) * D,), jnp.float32),
            pltpu.VMEM((L,), jnp.int32),
            pltpu.SemaphoreType.DMA,
        ],
    )


_segmax_128 = _make_segmax(128)
_segmax_256 = _make_segmax(256)


def _dot(a, b):
    return jnp.dot(a, b, preferred_element_type=jnp.float32, precision=_HIGH)


def _tc_pre_body(x_ref, w_ref, o_ref):
    o_ref[...] = _dot(x_ref[...], w_ref[...])


def _tc_pre(x, wT):
    return pl.pallas_call(
        _tc_pre_body,
        out_shape=jax.ShapeDtypeStruct((N, wT.shape[1]), jnp.float32),
    )(x, wT)


RB = 2000


def _dense_block(x, y, sg, w1x, w1a, b1, g1, bb1, w2, b2, be):
    agg = jnp.where(jnp.isfinite(sg), sg - y + be, 0.0)
    h = _dot(x, w1x) + _dot(agg, w1a) + b1
    mu = jnp.mean(h, axis=-1, keepdims=True)
    var = jnp.mean((h - mu) ** 2, axis=-1, keepdims=True)
    h = (h - mu) / jnp.sqrt(var + 1e-5) * g1 + bb1
    h = jnp.maximum(h, 0.0)
    return _dot(h, w2) + b2


def _tc_mid_body(x_ref, y_ref, sg_ref, w1x, w1a, b1, g1, bb1, w2, b2, be, wen,
                 xn_ref, yn_ref):
    x2 = _dense_block(x_ref[...], y_ref[...], sg_ref[...], w1x[...], w1a[...],
                      b1[...], g1[...], bb1[...], w2[...], b2[...], be[...])
    xn_ref[...] = x2
    yn_ref[...] = _dot(x2, wen[...])


def _row_spec(d):
    return pl.BlockSpec((RB, d), lambda i: (i, 0))


def _full_spec(shape):
    nd = len(shape)
    return pl.BlockSpec(shape, lambda i: (0,) * nd)


def _tc_mid(x, y, sg, w1x, w1a, b1, g1, bb1, w2, b2, be, wen):
    dx, dy, dn = x.shape[1], y.shape[1], wen.shape[1]
    args = (x, y, sg, w1x, w1a, b1, g1, bb1, w2, b2, be, wen)
    specs = [_row_spec(dx), _row_spec(dy), _row_spec(dy)] + [
        _full_spec(a.shape) for a in args[3:]
    ]
    return pl.pallas_call(
        _tc_mid_body,
        grid=(N // RB,),
        in_specs=specs,
        out_specs=[_row_spec(dy), _row_spec(dn)],
        out_shape=[
            jax.ShapeDtypeStruct((N, dy), jnp.float32),
            jax.ShapeDtypeStruct((N, dn), jnp.float32),
        ],
    )(*args)


def _tc_final_body(x_ref, y_ref, sg_ref, pen_ref, w1x, w1a, b1, g1, bb1, w2,
                   b2, be, o_ref):
    step = pl.program_id(0)
    x2 = _dense_block(x_ref[...], y_ref[...], sg_ref[...], w1x[...], w1a[...],
                      b1[...], g1[...], bb1[...], w2[...], b2[...], be[...])
    rows = []
    for g in range(NUM_GRAPHS):
        pg = jax.lax.broadcast_in_dim(pen_ref[:, g:g + 1], x2.shape, (0, 1))
        rows.append(jnp.max(x2 + pg, axis=0, keepdims=True))
    c = jnp.concatenate(rows, axis=0)
    prev = jnp.where(step == 0, jnp.full_like(c, -jnp.inf), o_ref[...])
    v = jnp.maximum(prev, c)
    v = jnp.where(step == N // RB - 1,
                  jnp.where(jnp.isfinite(v), v, 0.0), v)
    o_ref[...] = v


def _tc_final(x, y, sg, pen, w1x, w1a, b1, g1, bb1, w2, b2, be):
    dx, dy = x.shape[1], y.shape[1]
    args = (x, y, sg, pen, w1x, w1a, b1, g1, bb1, w2, b2, be)
    specs = [_row_spec(dx), _row_spec(dy), _row_spec(dy),
             _row_spec(NUM_GRAPHS)] + [
        _full_spec(a.shape) for a in args[4:]
    ]
    return pl.pallas_call(
        _tc_final_body,
        grid=(N // RB,),
        in_specs=specs,
        out_specs=pl.BlockSpec((NUM_GRAPHS, dy), lambda i: (0, 0)),
        out_shape=jax.ShapeDtypeStruct((NUM_GRAPHS, dy), jnp.float32),
    )(*args)


def _layer_weights(p):
    din = p["W1"].shape[1] - p["W1"].shape[0]
    w1x = p["W1"][:, :din].T
    w1a = p["W1"][:, din:].T
    return (w1x, w1a, p["b1"][None, :], p["g1"][None, :], p["bb1"][None, :],
            p["W2"].T, p["b2"][None, :], p["be"][None, :])


def kernel(edge_index, vertex_features, batch, params):
    x = vertex_features
    src = edge_index[0]
    dst = edge_index[1]
    p0, p1, p2 = params

    srcl, ldstl, cnts = _bucket(src, dst)

    y1 = _tc_pre(x, p0["We"].T)
    seg1 = _segmax_128(y1, srcl, ldstl, cnts).reshape(NPAD, 128)[:N]
    x2, y2 = _tc_mid(x, y1, seg1, *_layer_weights(p0), p1["We"].T)

    seg2 = _segmax_256(y2, srcl, ldstl, cnts).reshape(NPAD, 256)[:N]
    x3, y3 = _tc_mid(x2, y2, seg2, *_layer_weights(p1), p2["We"].T)

    seg3 = _segmax_256(y3, srcl, ldstl, cnts).reshape(NPAD, 256)[:N]
    pen = jnp.where(batch[:, None] == jnp.arange(NUM_GRAPHS)[None, :],
                    0.0, -jnp.inf).astype(jnp.float32)
    return _tc_final(x3, y3, seg3, pen, *_layer_weights(p2))


# speedup vs baseline: 2.5059x; 1.1563x over previous
import jax
import jax.numpy as jnp
from jax import lax
from jax.experimental import pallas as pl
from jax.experimental.pallas import tpu as pltpu
from jax.experimental.pallas import tpu_sc as plsc

N = 10000
E = 320000
NUM_GRAPHS = 16

NC, NS, L = 2, 16, 16
NW = NC * NS
NRANGE = 313
NPAD = NW * NRANGE
SENT = NRANGE
CH = 8000
NCH = E // CH
FBUF = CH + 128
RLEN = E + CH + 512

G = 64
SBN = 16
SBW = G * SBN
SW = 32

_MESH = plsc.VectorSubcoreMesh(
    core_axis_name="c", subcore_axis_name="s", num_cores=NC, num_subcores=NS
)
_SC_PARAMS = pltpu.CompilerParams(needs_layout_passes=False)

_HIGH = jax.lax.Precision.HIGHEST


def _wid():
    return lax.axis_index("s") * NC + lax.axis_index("c")


def _bucket_body(src_hbm, dst_hbm, srcl, ldstl, cnts, s_st, d_st, fsrc, fldst,
                 cnt_v, sem_st, sem_fl):
    wid = _wid()
    lo = wid * NRANGE
    hi = jnp.minimum(lo + NRANGE, N)
    base = wid * RLEN
    lanes = jnp.arange(L, dtype=jnp.int32)
    w1 = hi - lo - 1

    def fill_src(i, _):
        fsrc[pl.ds(i * L, L)] = jnp.zeros((L,), jnp.int32)
        return 0

    lax.fori_loop(0, FBUF // L, fill_src, 0)

    pltpu.sync_copy(src_hbm.at[pl.ds(0, CH)], s_st.at[pl.ds(0, CH)])
    pltpu.sync_copy(dst_hbm.at[pl.ds(0, CH)], d_st.at[pl.ds(0, CH)])

    def chunk(ci, total):
        par = (ci % 2) * CH
        parn = ((ci + 1) % 2) * CH

        @pl.when(ci > 0)
        def _():
            pltpu.make_async_copy(src_hbm.at[pl.ds(0, CH)],
                                  s_st.at[pl.ds(0, CH)], sem_st).wait()
            pltpu.make_async_copy(dst_hbm.at[pl.ds(0, CH)],
                                  d_st.at[pl.ds(0, CH)], sem_st).wait()

        @pl.when(ci + 1 < NCH)
        def _():
            nxt = pl.multiple_of((ci + 1) * CH, 8)
            pltpu.async_copy(src_hbm.at[pl.ds(nxt, CH)],
                             s_st.at[pl.ds(parn, CH)], sem_st)
            pltpu.async_copy(dst_hbm.at[pl.ds(nxt, CH)],
                             d_st.at[pl.ds(parn, CH)], sem_st)

        @pl.when(ci > 0)
        def _():
            pltpu.make_async_copy(fsrc.at[pl.ds(0, FBUF)],
                                  srcl.at[pl.ds(0, FBUF)], sem_fl).wait()
            pltpu.make_async_copy(fldst.at[pl.ds(0, FBUF)],
                                  ldstl.at[pl.ds(0, FBUF)], sem_fl).wait()

        def fill_ld(i, _):
            fldst[pl.ds(i * L, L)] = jnp.full((L,), SENT, jnp.int32)
            return 0

        lax.fori_loop(0, FBUF // L, fill_ld, 0)

        def scan16(j, offv):
            d = d_st[pl.ds(par + j * L, L)]
            s = s_st[pl.ds(par + j * L, L)]
            u = d - lo
            gg = jnp.minimum(u, w1 - u)
            inr = jnp.maximum(jnp.minimum(gg, 0) + 1, 0)
            cs = plsc.cumsum(inr)
            pos = inr * (offv + cs - 1) + (1 - inr) * (FBUF + lanes)
            plsc.store_scatter(fsrc, [pos], s)
            plsc.store_scatter(fldst, [pos], inr * u + (1 - inr) * SENT)
            cnt = plsc.all_reduce_population_count(inr.astype(bool))
            return offv + cnt

        offv = lax.fori_loop(0, CH // L, scan16, jnp.zeros((L,), jnp.int32))
        off = offv[0]
        fo = pl.multiple_of(base + total, 8)
        pltpu.async_copy(fsrc.at[pl.ds(0, FBUF)], srcl.at[pl.ds(fo, FBUF)],
                         sem_fl)
        pltpu.async_copy(fldst.at[pl.ds(0, FBUF)], ldstl.at[pl.ds(fo, FBUF)],
                         sem_fl)
        return total + ((off + 7) // 8) * 8

    total = lax.fori_loop(0, NCH, chunk, jnp.int32(0))
    pltpu.make_async_copy(fsrc.at[pl.ds(0, FBUF)],
                          srcl.at[pl.ds(0, FBUF)], sem_fl).wait()
    pltpu.make_async_copy(fldst.at[pl.ds(0, FBUF)],
                          ldstl.at[pl.ds(0, FBUF)], sem_fl).wait()
    kpad = ((total + 127) // 128) * 128
    cnt_v[...] = jnp.full((L,), 0, jnp.int32) + kpad
    pltpu.sync_copy(cnt_v, cnts.at[pl.ds(pl.multiple_of(wid * L, 8), L)])


_bucket = pl.kernel(
    _bucket_body,
    out_type=(
        jax.ShapeDtypeStruct((NW * RLEN,), jnp.int32),
        jax.ShapeDtypeStruct((NW * RLEN,), jnp.int32),
        jax.ShapeDtypeStruct((NW * L,), jnp.int32),
    ),
    mesh=_MESH,
    compiler_params=_SC_PARAMS,
    scratch_types=[
        pltpu.VMEM((2 * CH,), jnp.int32),
        pltpu.VMEM((2 * CH,), jnp.int32),
        pltpu.VMEM((FBUF + L,), jnp.int32),
        pltpu.VMEM((FBUF + L,), jnp.int32),
        pltpu.VMEM((L,), jnp.int32),
        pltpu.SemaphoreType.DMA,
        pltpu.SemaphoreType.DMA,
    ],
)


def _make_segmax(D):
    NSP = D // SW

    def body(*refs):
        y, srcl, ldstl, cnts = refs[:4]
        segout = refs[4:NSP + 4]
        rows = refs[NSP + 4]
        segs = refs[NSP + 5:2 * NSP + 5]
        idx_st, ldst_st, cnt_v, sem_g = refs[2 * NSP + 5:]

        wid = _wid()
        base = wid * RLEN
        lanes = jnp.arange(L, dtype=jnp.int32)
        neg = jnp.full((L,), -jnp.inf, jnp.float32)

        for k in range(NSP):
            def ini(t, _, k=k):
                segs[k][pl.ds(t * L, L)] = neg
                return 0

            lax.fori_loop(0, (NRANGE + 1) * SW // L, ini, 0)

        pltpu.sync_copy(cnts.at[pl.ds(pl.multiple_of(wid * L, 8), L)], cnt_v)
        nb = jnp.max(cnt_v[...]) // G

        def fire(b, sp_k):
            sp, k = sp_k
            io = sp * SBW + k * G
            gpar = (k % 2) * G
            pltpu.async_copy(y.at[idx_st.at[pl.ds(io, G)]],
                             rows.at[pl.ds(gpar, G), :], sem_g)

        def compute(ldoff, kprev):
            gpar = (kprev % 2) * G
            pltpu.make_async_copy(
                y.at[idx_st.at[pl.ds(0, G)]],
                rows.at[pl.ds(0, G), :], sem_g).wait()

            def edge(g, _):
                ld = ldst_st[pl.ds(ldoff + g, L)][0]
                sb = ld * SW
                gs = jnp.zeros((L,), jnp.int32) + (gpar + g)
                for j in range(NSP):
                    for c in range(SW // L):
                        v = plsc.load_gather(rows,
                                             [gs, lanes + j * SW + c * L])
                        segs[j][pl.ds(sb + c * L, L)] = jnp.maximum(
                            segs[j][pl.ds(sb + c * L, L)], v)
                return 0

            lax.fori_loop(0, G, edge, 0)

        def superbatch(s, _):
            sp = s % 2
            so = pl.multiple_of(base + s * SBW, 8)
            pltpu.sync_copy(srcl.at[pl.ds(so, SBW)],
                            idx_st.at[pl.ds(sp * SBW, SBW)])
            pltpu.sync_copy(ldstl.at[pl.ds(so, SBW)],
                            ldst_st.at[pl.ds(sp * SBW, SBW)])
            for k in range(SBN):
                b = s * SBN + k

                @pl.when(b < nb)
                def _(k=k, b=b, sp=sp):
                    fire(b, (sp, k))

                @pl.when((b >= 1) & (b < nb))
                def _(k=k, sp=sp, s=s):
                    if k == 0:
                        compute((1 - sp) * SBW + (SBN - 1) * G, SBN - 1)
                    else:
                        compute(sp * SBW + (k - 1) * G, k - 1)
            return 0

        nsb = (nb + SBN - 1) // SBN
        lax.fori_loop(0, nsb, superbatch, 0)

        @pl.when(nb > 0)
        def _():
            spl = ((nb - 1) // SBN) % 2
            kl = (nb - 1) % SBN
            compute(spl * SBW + kl * G, kl)

        for k in range(NSP):
            pltpu.sync_copy(
                segs[k].at[pl.ds(0, NRANGE * SW)],
                segout[k].at[pl.ds(pl.multiple_of(wid * (NRANGE * SW), 8),
                                   NRANGE * SW)])

    return pl.kernel(
        body,
        out_type=tuple(
            jax.ShapeDtypeStruct((NPAD * SW,), jnp.float32)
            for _ in range(NSP)),
        mesh=_MESH,
        compiler_params=_SC_PARAMS,
        scratch_types=(
            [pltpu.VMEM((2 * G, D), jnp.float32)]
            + [pltpu.VMEM(((NRANGE + 1) * SW,), jnp.float32)
               for _ in range(NSP)]
            + [pltpu.VMEM((2 * SBW,), jnp.int32),
               pltpu.VMEM((2 * SBW + L,), jnp.int32),
               pltpu.VMEM((L,), jnp.int32),
               pltpu.SemaphoreType.DMA]
        ),
    )


_segmax_128 = _make_segmax(128)
_segmax_256 = _make_segmax(256)


def _run_segmax(seg_fn, y, srcl, ldstl, cnts):
    outs = seg_fn(y, srcl, ldstl, cnts)
    return jnp.concatenate([o.reshape(NPAD, SW) for o in outs], axis=1)[:N]


def _dot(a, b):
    return jnp.dot(a, b, preferred_element_type=jnp.float32, precision=_HIGH)


def _tc_pre_body(x_ref, w_ref, o_ref):
    o_ref[...] = _dot(x_ref[...], w_ref[...])


def _tc_pre(x, wT):
    return pl.pallas_call(
        _tc_pre_body,
        out_shape=jax.ShapeDtypeStruct((N, wT.shape[1]), jnp.float32),
    )(x, wT)


RB = 2000


def _dense_block(x, y, sg, w1x, w1a, b1, g1, bb1, w2, b2, be):
    agg = jnp.where(jnp.isfinite(sg), sg - y + be, 0.0)
    h = _dot(x, w1x) + _dot(agg, w1a) + b1
    mu = jnp.mean(h, axis=-1, keepdims=True)
    var = jnp.mean((h - mu) ** 2, axis=-1, keepdims=True)
    h = (h - mu) / jnp.sqrt(var + 1e-5) * g1 + bb1
    h = jnp.maximum(h, 0.0)
    return _dot(h, w2) + b2


def _tc_mid_body(x_ref, y_ref, sg_ref, w1x, w1a, b1, g1, bb1, w2, b2, be, wen,
                 xn_ref, yn_ref):
    x2 = _dense_block(x_ref[...], y_ref[...], sg_ref[...], w1x[...], w1a[...],
                      b1[...], g1[...], bb1[...], w2[...], b2[...], be[...])
    xn_ref[...] = x2
    yn_ref[...] = _dot(x2, wen[...])


def _row_spec(d):
    return pl.BlockSpec((RB, d), lambda i: (i, 0))


def _full_spec(shape):
    nd = len(shape)
    return pl.BlockSpec(shape, lambda i: (0,) * nd)


def _tc_mid(x, y, sg, w1x, w1a, b1, g1, bb1, w2, b2, be, wen):
    dx, dy, dn = x.shape[1], y.shape[1], wen.shape[1]
    args = (x, y, sg, w1x, w1a, b1, g1, bb1, w2, b2, be, wen)
    specs = [_row_spec(dx), _row_spec(dy), _row_spec(dy)] + [
        _full_spec(a.shape) for a in args[3:]
    ]
    return pl.pallas_call(
        _tc_mid_body,
        grid=(N // RB,),
        in_specs=specs,
        out_specs=[_row_spec(dy), _row_spec(dn)],
        out_shape=[
            jax.ShapeDtypeStruct((N, dy), jnp.float32),
            jax.ShapeDtypeStruct((N, dn), jnp.float32),
        ],
    )(*args)


def _tc_final_body(x_ref, y_ref, sg_ref, pen_ref, w1x, w1a, b1, g1, bb1, w2,
                   b2, be, o_ref):
    step = pl.program_id(0)
    x2 = _dense_block(x_ref[...], y_ref[...], sg_ref[...], w1x[...], w1a[...],
                      b1[...], g1[...], bb1[...], w2[...], b2[...], be[...])
    rows = []
    for g in range(NUM_GRAPHS):
        pg = jax.lax.broadcast_in_dim(pen_ref[:, g:g + 1], x2.shape, (0, 1))
        rows.append(jnp.max(x2 + pg, axis=0, keepdims=True))
    c = jnp.concatenate(rows, axis=0)
    prev = jnp.where(step == 0, jnp.full_like(c, -jnp.inf), o_ref[...])
    v = jnp.maximum(prev, c)
    v = jnp.where(step == N // RB - 1,
                  jnp.where(jnp.isfinite(v), v, 0.0), v)
    o_ref[...] = v


def _tc_final(x, y, sg, pen, w1x, w1a, b1, g1, bb1, w2, b2, be):
    dx, dy = x.shape[1], y.shape[1]
    args = (x, y, sg, pen, w1x, w1a, b1, g1, bb1, w2, b2, be)
    specs = [_row_spec(dx), _row_spec(dy), _row_spec(dy),
             _row_spec(NUM_GRAPHS)] + [
        _full_spec(a.shape) for a in args[4:]
    ]
    return pl.pallas_call(
        _tc_final_body,
        grid=(N // RB,),
        in_specs=specs,
        out_specs=pl.BlockSpec((NUM_GRAPHS, dy), lambda i: (0, 0)),
        out_shape=jax.ShapeDtypeStruct((NUM_GRAPHS, dy), jnp.float32),
    )(*args)


def _layer_weights(p):
    din = p["W1"].shape[1] - p["W1"].shape[0]
    w1x = p["W1"][:, :din].T
    w1a = p["W1"][:, din:].T
    return (w1x, w1a, p["b1"][None, :], p["g1"][None, :], p["bb1"][None, :],
            p["W2"].T, p["b2"][None, :], p["be"][None, :])


def kernel(edge_index, vertex_features, batch, params):
    x = vertex_features
    p0, p1, p2 = params

    srcl, ldstl, cnts = _bucket(edge_index[0], edge_index[1])

    y1 = _tc_pre(x, p0["We"].T)
    seg1 = _run_segmax(_segmax_128, y1, srcl, ldstl, cnts)
    x2, y2 = _tc_mid(x, y1, seg1, *_layer_weights(p0), p1["We"].T)

    seg2 = _run_segmax(_segmax_256, y2, srcl, ldstl, cnts)
    x3, y3 = _tc_mid(x2, y2, seg2, *_layer_weights(p1), p2["We"].T)

    seg3 = _run_segmax(_segmax_256, y3, srcl, ldstl, cnts)
    pen = jnp.where(batch[:, None] == jnp.arange(NUM_GRAPHS)[None, :],
                    0.0, -jnp.inf).astype(jnp.float32)
    return _tc_final(x3, y3, seg3, pen, *_layer_weights(p2))


# speedup vs baseline: 4.5356x; 1.8100x over previous
import jax
import jax.numpy as jnp
from jax import lax
from jax.experimental import pallas as pl
from jax.experimental.pallas import tpu as pltpu
from jax.experimental.pallas import tpu_sc as plsc

N = 10000
E = 320000
NUM_GRAPHS = 16

NC, NS, L = 2, 16, 16
NW = NC * NS
NRANGE = 313
NPAD = NW * NRANGE
SENT = NRANGE
CH = 8000
NCH = E // CH
FBUF = CH + 128
RLEN = E + CH + 512

G = 64
SBN = 16
SBW = G * SBN
SW = 32

_MESH = plsc.VectorSubcoreMesh(
    core_axis_name="c", subcore_axis_name="s", num_cores=NC, num_subcores=NS
)
_SC_PARAMS = pltpu.CompilerParams(needs_layout_passes=False)

_HIGH = jax.lax.Precision.HIGHEST


def _wid():
    return lax.axis_index("s") * NC + lax.axis_index("c")


def _bucket_body(src_hbm, dst_hbm, srcl, ldstl, cnts, s_st, d_st, fsrc, fldst,
                 cnt_v, sem_st, sem_fl):
    wid = _wid()
    lo = wid * NRANGE
    hi = jnp.minimum(lo + NRANGE, N)
    base = wid * RLEN
    lanes = jnp.arange(L, dtype=jnp.int32)
    w1 = hi - lo - 1

    def fill_src(i, _):
        fsrc[pl.ds(i * L, L)] = jnp.zeros((L,), jnp.int32)
        return 0

    lax.fori_loop(0, FBUF // L, fill_src, 0)

    pltpu.sync_copy(src_hbm.at[pl.ds(0, CH)], s_st.at[pl.ds(0, CH)])
    pltpu.sync_copy(dst_hbm.at[pl.ds(0, CH)], d_st.at[pl.ds(0, CH)])

    def chunk(ci, total):
        par = (ci % 2) * CH
        parn = ((ci + 1) % 2) * CH

        @pl.when(ci > 0)
        def _():
            pltpu.make_async_copy(src_hbm.at[pl.ds(0, CH)],
                                  s_st.at[pl.ds(0, CH)], sem_st).wait()
            pltpu.make_async_copy(dst_hbm.at[pl.ds(0, CH)],
                                  d_st.at[pl.ds(0, CH)], sem_st).wait()

        @pl.when(ci + 1 < NCH)
        def _():
            nxt = pl.multiple_of((ci + 1) * CH, 8)
            pltpu.async_copy(src_hbm.at[pl.ds(nxt, CH)],
                             s_st.at[pl.ds(parn, CH)], sem_st)
            pltpu.async_copy(dst_hbm.at[pl.ds(nxt, CH)],
                             d_st.at[pl.ds(parn, CH)], sem_st)

        @pl.when(ci > 0)
        def _():
            pltpu.make_async_copy(fsrc.at[pl.ds(0, FBUF)],
                                  srcl.at[pl.ds(0, FBUF)], sem_fl).wait()
            pltpu.make_async_copy(fldst.at[pl.ds(0, FBUF)],
                                  ldstl.at[pl.ds(0, FBUF)], sem_fl).wait()

        def fill_ld(i, _):
            fldst[pl.ds(i * L, L)] = jnp.full((L,), SENT, jnp.int32)
            return 0

        lax.fori_loop(0, FBUF // L, fill_ld, 0)

        def scan16(j, offv):
            d = d_st[pl.ds(par + j * L, L)]
            s = s_st[pl.ds(par + j * L, L)]
            u = d - lo
            gg = jnp.minimum(u, w1 - u)
            inr = jnp.maximum(jnp.minimum(gg, 0) + 1, 0)
            cs = plsc.cumsum(inr)
            pos = inr * (offv + cs - 1) + (1 - inr) * (FBUF + lanes)
            plsc.store_scatter(fsrc, [pos], s)
            plsc.store_scatter(fldst, [pos], inr * u + (1 - inr) * SENT)
            cnt = plsc.all_reduce_population_count(inr.astype(bool))
            return offv + cnt

        offv = lax.fori_loop(0, CH // L, scan16, jnp.zeros((L,), jnp.int32))
        off = offv[0]
        fo = pl.multiple_of(base + total, 8)
        pltpu.async_copy(fsrc.at[pl.ds(0, FBUF)], srcl.at[pl.ds(fo, FBUF)],
                         sem_fl)
        pltpu.async_copy(fldst.at[pl.ds(0, FBUF)], ldstl.at[pl.ds(fo, FBUF)],
                         sem_fl)
        return total + ((off + 7) // 8) * 8

    total = lax.fori_loop(0, NCH, chunk, jnp.int32(0))
    pltpu.make_async_copy(fsrc.at[pl.ds(0, FBUF)],
                          srcl.at[pl.ds(0, FBUF)], sem_fl).wait()
    pltpu.make_async_copy(fldst.at[pl.ds(0, FBUF)],
                          ldstl.at[pl.ds(0, FBUF)], sem_fl).wait()
    kpad = ((total + 127) // 128) * 128
    cnt_v[...] = jnp.full((L,), 0, jnp.int32) + kpad
    pltpu.sync_copy(cnt_v, cnts.at[pl.ds(pl.multiple_of(wid * L, 8), L)])


_bucket = pl.kernel(
    _bucket_body,
    out_type=(
        jax.ShapeDtypeStruct((NW * RLEN,), jnp.int32),
        jax.ShapeDtypeStruct((NW * RLEN,), jnp.int32),
        jax.ShapeDtypeStruct((NW * L,), jnp.int32),
    ),
    mesh=_MESH,
    compiler_params=_SC_PARAMS,
    scratch_types=[
        pltpu.VMEM((2 * CH,), jnp.int32),
        pltpu.VMEM((2 * CH,), jnp.int32),
        pltpu.VMEM((FBUF + L,), jnp.int32),
        pltpu.VMEM((FBUF + L,), jnp.int32),
        pltpu.VMEM((L,), jnp.int32),
        pltpu.SemaphoreType.DMA,
        pltpu.SemaphoreType.DMA,
    ],
)


def _make_segmax(D):
    NSP = D // SW

    def body(*refs):
        y, srcl, ldstl, cnts = refs[:4]
        segout = refs[4:NSP + 4]
        rows = refs[NSP + 4]
        segs = refs[NSP + 5:2 * NSP + 5]
        idx_st, ldst_st, cnt_v, sem_g = refs[2 * NSP + 5:]

        wid = _wid()
        base = wid * RLEN
        lanes = jnp.arange(L, dtype=jnp.int32)
        neg = jnp.full((L,), -jnp.inf, jnp.float32)

        for k in range(NSP):
            def ini(t, _, k=k):
                segs[k][pl.ds(t * L, L)] = neg
                return 0

            lax.fori_loop(0, (NRANGE + 1) * SW // L, ini, 0)

        pltpu.sync_copy(cnts.at[pl.ds(pl.multiple_of(wid * L, 8), L)], cnt_v)
        nb = jnp.max(cnt_v[...]) // G

        def fire(b, sp_k):
            sp, k = sp_k
            io = sp * SBW + k * G
            gpar = (k % 2) * G
            pltpu.async_copy(y.at[idx_st.at[pl.ds(io, G)]],
                             rows.at[pl.ds(gpar, G), :], sem_g)

        def compute(ldoff, kprev):
            gpar = (kprev % 2) * G
            pltpu.make_async_copy(
                y.at[idx_st.at[pl.ds(0, G)]],
                rows.at[pl.ds(0, G), :], sem_g).wait()

            def edge(g, _):
                ld = ldst_st[pl.ds(ldoff + g, L)][0]
                sb = ld * SW
                gs = jnp.zeros((L,), jnp.int32) + (gpar + g)
                cols = [(j, c) for j in range(NSP) for c in range(SW // L)]
                vals = [plsc.load_gather(rows, [gs, lanes + j * SW + c * L])
                        for (j, c) in cols]
                olds = [segs[j][pl.ds(sb + c * L, L)] for (j, c) in cols]
                for t, (j, c) in enumerate(cols):
                    segs[j][pl.ds(sb + c * L, L)] = jnp.maximum(
                        olds[t], vals[t])
                return 0

            lax.fori_loop(0, G, edge, 0, unroll=2)

        def superbatch(s, _):
            sp = s % 2
            so = pl.multiple_of(base + s * SBW, 8)
            pltpu.sync_copy(srcl.at[pl.ds(so, SBW)],
                            idx_st.at[pl.ds(sp * SBW, SBW)])
            pltpu.sync_copy(ldstl.at[pl.ds(so, SBW)],
                            ldst_st.at[pl.ds(sp * SBW, SBW)])
            for k in range(SBN):
                b = s * SBN + k

                @pl.when(b < nb)
                def _(k=k, b=b, sp=sp):
                    fire(b, (sp, k))

                @pl.when((b >= 1) & (b < nb))
                def _(k=k, sp=sp, s=s):
                    if k == 0:
                        compute((1 - sp) * SBW + (SBN - 1) * G, SBN - 1)
                    else:
                        compute(sp * SBW + (k - 1) * G, k - 1)
            return 0

        nsb = (nb + SBN - 1) // SBN
        lax.fori_loop(0, nsb, superbatch, 0)

        @pl.when(nb > 0)
        def _():
            spl = ((nb - 1) // SBN) % 2
            kl = (nb - 1) % SBN
            compute(spl * SBW + kl * G, kl)

        for k in range(NSP):
            pltpu.sync_copy(
                segs[k].at[pl.ds(0, NRANGE * SW)],
                segout[k].at[pl.ds(pl.multiple_of(wid * (NRANGE * SW), 8),
                                   NRANGE * SW)])

    return pl.kernel(
        body,
        out_type=tuple(
            jax.ShapeDtypeStruct((NPAD * SW,), jnp.float32)
            for _ in range(NSP)),
        mesh=_MESH,
        compiler_params=_SC_PARAMS,
        scratch_types=(
            [pltpu.VMEM((2 * G, D), jnp.float32)]
            + [pltpu.VMEM(((NRANGE + 1) * SW,), jnp.float32)
               for _ in range(NSP)]
            + [pltpu.VMEM((2 * SBW,), jnp.int32),
               pltpu.VMEM((2 * SBW + L,), jnp.int32),
               pltpu.VMEM((L,), jnp.int32),
               pltpu.SemaphoreType.DMA]
        ),
    )


_segmax_128 = _make_segmax(128)
_segmax_256 = _make_segmax(256)


def _run_segmax(seg_fn, y, srcl, ldstl, cnts):
    outs = seg_fn(y, srcl, ldstl, cnts)
    return jnp.concatenate([o.reshape(NPAD, SW) for o in outs], axis=1)[:N]


def _dot(a, b):
    return jnp.dot(a, b, preferred_element_type=jnp.float32, precision=_HIGH)


def _tc_pre_body(x_ref, w_ref, o_ref):
    o_ref[...] = _dot(x_ref[...], w_ref[...])


def _tc_pre(x, wT):
    return pl.pallas_call(
        _tc_pre_body,
        out_shape=jax.ShapeDtypeStruct((N, wT.shape[1]), jnp.float32),
    )(x, wT)


RB = 2000


def _dense_block(x, y, sg, w1x, w1a, b1, g1, bb1, w2, b2, be):
    agg = jnp.where(jnp.isfinite(sg), sg - y + be, 0.0)
    h = _dot(x, w1x) + _dot(agg, w1a) + b1
    mu = jnp.mean(h, axis=-1, keepdims=True)
    var = jnp.mean((h - mu) ** 2, axis=-1, keepdims=True)
    h = (h - mu) / jnp.sqrt(var + 1e-5) * g1 + bb1
    h = jnp.maximum(h, 0.0)
    return _dot(h, w2) + b2


def _tc_mid_body(x_ref, y_ref, sg_ref, w1x, w1a, b1, g1, bb1, w2, b2, be, wen,
                 xn_ref, yn_ref):
    x2 = _dense_block(x_ref[...], y_ref[...], sg_ref[...], w1x[...], w1a[...],
                      b1[...], g1[...], bb1[...], w2[...], b2[...], be[...])
    xn_ref[...] = x2
    yn_ref[...] = _dot(x2, wen[...])


def _row_spec(d):
    return pl.BlockSpec((RB, d), lambda i: (i, 0))


def _full_spec(shape):
    nd = len(shape)
    return pl.BlockSpec(shape, lambda i: (0,) * nd)


def _tc_mid(x, y, sg, w1x, w1a, b1, g1, bb1, w2, b2, be, wen):
    dx, dy, dn = x.shape[1], y.shape[1], wen.shape[1]
    args = (x, y, sg, w1x, w1a, b1, g1, bb1, w2, b2, be, wen)
    specs = [_row_spec(dx), _row_spec(dy), _row_spec(dy)] + [
        _full_spec(a.shape) for a in args[3:]
    ]
    return pl.pallas_call(
        _tc_mid_body,
        grid=(N // RB,),
        in_specs=specs,
        out_specs=[_row_spec(dy), _row_spec(dn)],
        out_shape=[
            jax.ShapeDtypeStruct((N, dy), jnp.float32),
            jax.ShapeDtypeStruct((N, dn), jnp.float32),
        ],
    )(*args)


def _tc_final_body(x_ref, y_ref, sg_ref, pen_ref, w1x, w1a, b1, g1, bb1, w2,
                   b2, be, o_ref):
    step = pl.program_id(0)
    x2 = _dense_block(x_ref[...], y_ref[...], sg_ref[...], w1x[...], w1a[...],
                      b1[...], g1[...], bb1[...], w2[...], b2[...], be[...])
    rows = []
    for g in range(NUM_GRAPHS):
        pg = jax.lax.broadcast_in_dim(pen_ref[:, g:g + 1], x2.shape, (0, 1))
        rows.append(jnp.max(x2 + pg, axis=0, keepdims=True))
    c = jnp.concatenate(rows, axis=0)
    prev = jnp.where(step == 0, jnp.full_like(c, -jnp.inf), o_ref[...])
    v = jnp.maximum(prev, c)
    v = jnp.where(step == N // RB - 1,
                  jnp.where(jnp.isfinite(v), v, 0.0), v)
    o_ref[...] = v


def _tc_final(x, y, sg, pen, w1x, w1a, b1, g1, bb1, w2, b2, be):
    dx, dy = x.shape[1], y.shape[1]
    args = (x, y, sg, pen, w1x, w1a, b1, g1, bb1, w2, b2, be)
    specs = [_row_spec(dx), _row_spec(dy), _row_spec(dy),
             _row_spec(NUM_GRAPHS)] + [
        _full_spec(a.shape) for a in args[4:]
    ]
    return pl.pallas_call(
        _tc_final_body,
        grid=(N // RB,),
        in_specs=specs,
        out_specs=pl.BlockSpec((NUM_GRAPHS, dy), lambda i: (0, 0)),
        out_shape=jax.ShapeDtypeStruct((NUM_GRAPHS, dy), jnp.float32),
    )(*args)


def _layer_weights(p):
    din = p["W1"].shape[1] - p["W1"].shape[0]
    w1x = p["W1"][:, :din].T
    w1a = p["W1"][:, din:].T
    return (w1x, w1a, p["b1"][None, :], p["g1"][None, :], p["bb1"][None, :],
            p["W2"].T, p["b2"][None, :], p["be"][None, :])


def kernel(edge_index, vertex_features, batch, params):
    x = vertex_features
    p0, p1, p2 = params

    srcl, ldstl, cnts = _bucket(edge_index[0], edge_index[1])

    y1 = _tc_pre(x, p0["We"].T)
    seg1 = _run_segmax(_segmax_128, y1, srcl, ldstl, cnts)
    x2, y2 = _tc_mid(x, y1, seg1, *_layer_weights(p0), p1["We"].T)

    seg2 = _run_segmax(_segmax_256, y2, srcl, ldstl, cnts)
    x3, y3 = _tc_mid(x2, y2, seg2, *_layer_weights(p1), p2["We"].T)

    seg3 = _run_segmax(_segmax_256, y3, srcl, ldstl, cnts)
    pen = jnp.where(batch[:, None] == jnp.arange(NUM_GRAPHS)[None, :],
                    0.0, -jnp.inf).astype(jnp.float32)
    return _tc_final(x3, y3, seg3, pen, *_layer_weights(p2))


# speedup vs baseline: 5.1214x; 1.1291x over previous
import jax
import jax.numpy as jnp
from jax import lax
from jax.experimental import pallas as pl
from jax.experimental.pallas import tpu as pltpu
from jax.experimental.pallas import tpu_sc as plsc

N = 10000
E = 320000
NUM_GRAPHS = 16

NC, NS, L = 2, 16, 16
NW = NC * NS
NRANGE = 313
NPAD = NW * NRANGE
SENT = NRANGE
CH = 8000
NCH = E // CH
FBUF = CH + 128
RLEN = E + CH + 512

G = 64
SBN = 16
SBW = G * SBN
SW = 32

_MESH = plsc.VectorSubcoreMesh(
    core_axis_name="c", subcore_axis_name="s", num_cores=NC, num_subcores=NS
)
_SC_PARAMS = pltpu.CompilerParams(needs_layout_passes=False)

_HIGH = jax.lax.Precision.HIGHEST


def _wid():
    return lax.axis_index("s") * NC + lax.axis_index("c")


def _bucket_body(src_hbm, dst_hbm, srcl, ldstl, cnts, s_st, d_st, fsrc, fldst,
                 cnt_v, sem_st, sem_fl):
    wid = _wid()
    lo = wid * NRANGE
    hi = jnp.minimum(lo + NRANGE, N)
    base = wid * RLEN
    lanes = jnp.arange(L, dtype=jnp.int32)
    w1 = hi - lo - 1

    def fill_src(i, _):
        fsrc[pl.ds(i * L, L)] = jnp.zeros((L,), jnp.int32)
        return 0

    lax.fori_loop(0, FBUF // L, fill_src, 0)

    pltpu.sync_copy(src_hbm.at[pl.ds(0, CH)], s_st.at[pl.ds(0, CH)])
    pltpu.sync_copy(dst_hbm.at[pl.ds(0, CH)], d_st.at[pl.ds(0, CH)])

    def chunk(ci, total):
        par = (ci % 2) * CH
        parn = ((ci + 1) % 2) * CH

        @pl.when(ci > 0)
        def _():
            pltpu.make_async_copy(src_hbm.at[pl.ds(0, CH)],
                                  s_st.at[pl.ds(0, CH)], sem_st).wait()
            pltpu.make_async_copy(dst_hbm.at[pl.ds(0, CH)],
                                  d_st.at[pl.ds(0, CH)], sem_st).wait()

        @pl.when(ci + 1 < NCH)
        def _():
            nxt = pl.multiple_of((ci + 1) * CH, 8)
            pltpu.async_copy(src_hbm.at[pl.ds(nxt, CH)],
                             s_st.at[pl.ds(parn, CH)], sem_st)
            pltpu.async_copy(dst_hbm.at[pl.ds(nxt, CH)],
                             d_st.at[pl.ds(parn, CH)], sem_st)

        @pl.when(ci > 0)
        def _():
            pltpu.make_async_copy(fsrc.at[pl.ds(0, FBUF)],
                                  srcl.at[pl.ds(0, FBUF)], sem_fl).wait()
            pltpu.make_async_copy(fldst.at[pl.ds(0, FBUF)],
                                  ldstl.at[pl.ds(0, FBUF)], sem_fl).wait()

        def fill_ld(i, _):
            fldst[pl.ds(i * L, L)] = jnp.full((L,), SENT, jnp.int32)
            return 0

        lax.fori_loop(0, FBUF // L, fill_ld, 0)

        SU = 4

        def scan16(j, offv):
            us, ss, css, pcs = [], [], [], []
            for t in range(SU):
                d = d_st[pl.ds(par + (j * SU + t) * L, L)]
                ss.append(s_st[pl.ds(par + (j * SU + t) * L, L)])
                u = d - lo
                gg = jnp.minimum(u, w1 - u)
                inr = jnp.maximum(jnp.minimum(gg, 0) + 1, 0)
                us.append((u, inr))
                css.append(plsc.cumsum(inr))
                pcs.append(plsc.all_reduce_population_count(inr.astype(bool)))
            for t in range(SU):
                u, inr = us[t]
                pos = inr * (offv + css[t] - 1) + (1 - inr) * (FBUF + lanes)
                plsc.store_scatter(fsrc, [pos], ss[t])
                plsc.store_scatter(fldst, [pos], inr * u + (1 - inr) * SENT)
                offv = offv + pcs[t]
            return offv

        offv = lax.fori_loop(0, CH // (L * SU), scan16,
                             jnp.zeros((L,), jnp.int32))
        off = offv[0]
        fo = pl.multiple_of(base + total, 8)
        pltpu.async_copy(fsrc.at[pl.ds(0, FBUF)], srcl.at[pl.ds(fo, FBUF)],
                         sem_fl)
        pltpu.async_copy(fldst.at[pl.ds(0, FBUF)], ldstl.at[pl.ds(fo, FBUF)],
                         sem_fl)
        return total + ((off + 7) // 8) * 8

    total = lax.fori_loop(0, NCH, chunk, jnp.int32(0))
    pltpu.make_async_copy(fsrc.at[pl.ds(0, FBUF)],
                          srcl.at[pl.ds(0, FBUF)], sem_fl).wait()
    pltpu.make_async_copy(fldst.at[pl.ds(0, FBUF)],
                          ldstl.at[pl.ds(0, FBUF)], sem_fl).wait()
    kpad = ((total + 127) // 128) * 128
    cnt_v[...] = jnp.full((L,), 0, jnp.int32) + kpad
    pltpu.sync_copy(cnt_v, cnts.at[pl.ds(pl.multiple_of(wid * L, 8), L)])


_bucket = pl.kernel(
    _bucket_body,
    out_type=(
        jax.ShapeDtypeStruct((NW * RLEN,), jnp.int32),
        jax.ShapeDtypeStruct((NW * RLEN,), jnp.int32),
        jax.ShapeDtypeStruct((NW * L,), jnp.int32),
    ),
    mesh=_MESH,
    compiler_params=_SC_PARAMS,
    scratch_types=[
        pltpu.VMEM((2 * CH,), jnp.int32),
        pltpu.VMEM((2 * CH,), jnp.int32),
        pltpu.VMEM((FBUF + L,), jnp.int32),
        pltpu.VMEM((FBUF + L,), jnp.int32),
        pltpu.VMEM((L,), jnp.int32),
        pltpu.SemaphoreType.DMA,
        pltpu.SemaphoreType.DMA,
    ],
)


def _make_segmax(D):
    NSP = D // SW

    def body(*refs):
        y, srcl, ldstl, cnts = refs[:4]
        segout = refs[4:NSP + 4]
        rows = refs[NSP + 4]
        segs = refs[NSP + 5:2 * NSP + 5]
        idx_st, ldst_st, cnt_v, sem_g = refs[2 * NSP + 5:]

        wid = _wid()
        base = wid * RLEN
        lanes = jnp.arange(L, dtype=jnp.int32)
        neg = jnp.full((L,), -jnp.inf, jnp.float32)

        for k in range(NSP):
            def ini(t, _, k=k):
                segs[k][pl.ds(t * L, L)] = neg
                return 0

            lax.fori_loop(0, (NRANGE + 1) * SW // L, ini, 0)

        pltpu.sync_copy(cnts.at[pl.ds(pl.multiple_of(wid * L, 8), L)], cnt_v)
        nb = jnp.max(cnt_v[...]) // G

        def fire(b, sp_k):
            sp, k = sp_k
            io = sp * SBW + k * G
            gpar = (k % 2) * G
            pltpu.async_copy(y.at[idx_st.at[pl.ds(io, G)]],
                             rows.at[pl.ds(gpar, G), :], sem_g)

        def compute(ldoff, kprev):
            gpar = (kprev % 2) * G
            pltpu.make_async_copy(
                y.at[idx_st.at[pl.ds(0, G)]],
                rows.at[pl.ds(0, G), :], sem_g).wait()

            def edge(g, _):
                ld = ldst_st[pl.ds(ldoff + g, L)][0]
                sb = ld * SW
                gs = jnp.zeros((L,), jnp.int32) + (gpar + g)
                cols = [(j, c) for j in range(NSP) for c in range(SW // L)]
                vals = [plsc.load_gather(rows, [gs, lanes + j * SW + c * L])
                        for (j, c) in cols]
                olds = [segs[j][pl.ds(sb + c * L, L)] for (j, c) in cols]
                for t, (j, c) in enumerate(cols):
                    segs[j][pl.ds(sb + c * L, L)] = jnp.maximum(
                        olds[t], vals[t])
                return 0

            lax.fori_loop(0, G, edge, 0, unroll=2)

        def superbatch(s, _):
            sp = s % 2
            so = pl.multiple_of(base + s * SBW, 8)
            pltpu.sync_copy(srcl.at[pl.ds(so, SBW)],
                            idx_st.at[pl.ds(sp * SBW, SBW)])
            pltpu.sync_copy(ldstl.at[pl.ds(so, SBW)],
                            ldst_st.at[pl.ds(sp * SBW, SBW)])
            for k in range(SBN):
                b = s * SBN + k

                @pl.when(b < nb)
                def _(k=k, b=b, sp=sp):
                    fire(b, (sp, k))

                @pl.when((b >= 1) & (b < nb))
                def _(k=k, sp=sp, s=s):
                    if k == 0:
                        compute((1 - sp) * SBW + (SBN - 1) * G, SBN - 1)
                    else:
                        compute(sp * SBW + (k - 1) * G, k - 1)
            return 0

        nsb = (nb + SBN - 1) // SBN
        lax.fori_loop(0, nsb, superbatch, 0)

        @pl.when(nb > 0)
        def _():
            spl = ((nb - 1) // SBN) % 2
            kl = (nb - 1) % SBN
            compute(spl * SBW + kl * G, kl)

        for k in range(NSP):
            pltpu.sync_copy(
                segs[k].at[pl.ds(0, NRANGE * SW)],
                segout[k].at[pl.ds(pl.multiple_of(wid * (NRANGE * SW), 8),
                                   NRANGE * SW)])

    return pl.kernel(
        body,
        out_type=tuple(
            jax.ShapeDtypeStruct((NPAD * SW,), jnp.float32)
            for _ in range(NSP)),
        mesh=_MESH,
        compiler_params=_SC_PARAMS,
        scratch_types=(
            [pltpu.VMEM((2 * G, D), jnp.float32)]
            + [pltpu.VMEM(((NRANGE + 1) * SW,), jnp.float32)
               for _ in range(NSP)]
            + [pltpu.VMEM((2 * SBW,), jnp.int32),
               pltpu.VMEM((2 * SBW + L,), jnp.int32),
               pltpu.VMEM((L,), jnp.int32),
               pltpu.SemaphoreType.DMA]
        ),
    )


_segmax_128 = _make_segmax(128)
_segmax_256 = _make_segmax(256)


def _run_segmax(seg_fn, y, srcl, ldstl, cnts):
    outs = seg_fn(y, srcl, ldstl, cnts)
    return jnp.concatenate([o.reshape(NPAD, SW) for o in outs], axis=1)[:N]


def _dot(a, b):
    return jnp.dot(a, b, preferred_element_type=jnp.float32, precision=_HIGH)


def _tc_pre_body(x_ref, w_ref, o_ref):
    o_ref[...] = _dot(x_ref[...], w_ref[...])


def _tc_pre(x, wT):
    return pl.pallas_call(
        _tc_pre_body,
        out_shape=jax.ShapeDtypeStruct((N, wT.shape[1]), jnp.float32),
    )(x, wT)


RB = 2000


def _dense_block(x, y, sg, w1x, w1a, b1, g1, bb1, w2, b2, be):
    agg = jnp.where(jnp.isfinite(sg), sg - y + be, 0.0)
    h = _dot(x, w1x) + _dot(agg, w1a) + b1
    mu = jnp.mean(h, axis=-1, keepdims=True)
    var = jnp.mean((h - mu) ** 2, axis=-1, keepdims=True)
    h = (h - mu) / jnp.sqrt(var + 1e-5) * g1 + bb1
    h = jnp.maximum(h, 0.0)
    return _dot(h, w2) + b2


def _tc_mid_body(x_ref, y_ref, sg_ref, w1x, w1a, b1, g1, bb1, w2, b2, be, wen,
                 xn_ref, yn_ref):
    x2 = _dense_block(x_ref[...], y_ref[...], sg_ref[...], w1x[...], w1a[...],
                      b1[...], g1[...], bb1[...], w2[...], b2[...], be[...])
    xn_ref[...] = x2
    yn_ref[...] = _dot(x2, wen[...])


def _row_spec(d):
    return pl.BlockSpec((RB, d), lambda i: (i, 0))


def _full_spec(shape):
    nd = len(shape)
    return pl.BlockSpec(shape, lambda i: (0,) * nd)


def _tc_mid(x, y, sg, w1x, w1a, b1, g1, bb1, w2, b2, be, wen):
    dx, dy, dn = x.shape[1], y.shape[1], wen.shape[1]
    args = (x, y, sg, w1x, w1a, b1, g1, bb1, w2, b2, be, wen)
    specs = [_row_spec(dx), _row_spec(dy), _row_spec(dy)] + [
        _full_spec(a.shape) for a in args[3:]
    ]
    return pl.pallas_call(
        _tc_mid_body,
        grid=(N // RB,),
        in_specs=specs,
        out_specs=[_row_spec(dy), _row_spec(dn)],
        out_shape=[
            jax.ShapeDtypeStruct((N, dy), jnp.float32),
            jax.ShapeDtypeStruct((N, dn), jnp.float32),
        ],
    )(*args)


def _tc_final_body(x_ref, y_ref, sg_ref, pen_ref, w1x, w1a, b1, g1, bb1, w2,
                   b2, be, o_ref):
    step = pl.program_id(0)
    x2 = _dense_block(x_ref[...], y_ref[...], sg_ref[...], w1x[...], w1a[...],
                      b1[...], g1[...], bb1[...], w2[...], b2[...], be[...])
    rows = []
    for g in range(NUM_GRAPHS):
        pg = jax.lax.broadcast_in_dim(pen_ref[:, g:g + 1], x2.shape, (0, 1))
        rows.append(jnp.max(x2 + pg, axis=0, keepdims=True))
    c = jnp.concatenate(rows, axis=0)
    prev = jnp.where(step == 0, jnp.full_like(c, -jnp.inf), o_ref[...])
    v = jnp.maximum(prev, c)
    v = jnp.where(step == N // RB - 1,
                  jnp.where(jnp.isfinite(v), v, 0.0), v)
    o_ref[...] = v


def _tc_final(x, y, sg, pen, w1x, w1a, b1, g1, bb1, w2, b2, be):
    dx, dy = x.shape[1], y.shape[1]
    args = (x, y, sg, pen, w1x, w1a, b1, g1, bb1, w2, b2, be)
    specs = [_row_spec(dx), _row_spec(dy), _row_spec(dy),
             _row_spec(NUM_GRAPHS)] + [
        _full_spec(a.shape) for a in args[4:]
    ]
    return pl.pallas_call(
        _tc_final_body,
        grid=(N // RB,),
        in_specs=specs,
        out_specs=pl.BlockSpec((NUM_GRAPHS, dy), lambda i: (0, 0)),
        out_shape=jax.ShapeDtypeStruct((NUM_GRAPHS, dy), jnp.float32),
    )(*args)


def _layer_weights(p):
    din = p["W1"].shape[1] - p["W1"].shape[0]
    w1x = p["W1"][:, :din].T
    w1a = p["W1"][:, din:].T
    return (w1x, w1a, p["b1"][None, :], p["g1"][None, :], p["bb1"][None, :],
            p["W2"].T, p["b2"][None, :], p["be"][None, :])


def kernel(edge_index, vertex_features, batch, params):
    x = vertex_features
    p0, p1, p2 = params

    srcl, ldstl, cnts = _bucket(edge_index[0], edge_index[1])

    y1 = _tc_pre(x, p0["We"].T)
    seg1 = _run_segmax(_segmax_128, y1, srcl, ldstl, cnts)
    x2, y2 = _tc_mid(x, y1, seg1, *_layer_weights(p0), p1["We"].T)

    seg2 = _run_segmax(_segmax_256, y2, srcl, ldstl, cnts)
    x3, y3 = _tc_mid(x2, y2, seg2, *_layer_weights(p1), p2["We"].T)

    seg3 = _run_segmax(_segmax_256, y3, srcl, ldstl, cnts)
    pen = jnp.where(batch[:, None] == jnp.arange(NUM_GRAPHS)[None, :],
                    0.0, -jnp.inf).astype(jnp.float32)
    return _tc_final(x3, y3, seg3, pen, *_layer_weights(p2))
